# baseline (speedup 1.0000x reference)
import functools

import jax
import jax.numpy as jnp
from jax import lax
from jax.experimental import pallas as pl
from jax.experimental.pallas import tpu as pltpu
from jax.experimental.pallas import tpu_sc as plsc

_B, _C, _N, _K, _G = 2, 144, 16384, 16, 9
_D = _C // _G
_ROWW = 2 * _C
_NPTS = _B * _N
_NC, _NS, _L = 2, 16, 16
_NW = _NC * _NS
_PPW = _NPTS // _NW
_CHUNK = 8
_NCH = _PPW // _CHUNK
_IDXN = _CHUNK * _K
_BCH = 16
_NBLK = _NCH // _BCH


def _perm(vec, idx_const):
    dnums = lax.GatherDimensionNumbers(
        offset_dims=(), collapsed_slice_dims=(0,), start_index_map=(0,))
    return lax.gather(vec, idx_const[:, None], dnums, (1,),
                      mode=lax.GatherScatterMode.PROMISE_IN_BOUNDS)


def _tree_max(v, perms):
    for idxc in perms:
        v = jnp.maximum(v, _perm(v, idxc))
    return v


def _tree_sum(v, perms):
    for idxc in perms:
        v = v + _perm(v, idxc)
    return v


def _bcast_lane(vec, lane):
    idx = jnp.full((_L, 1), lane, jnp.int32)
    dnums = lax.GatherDimensionNumbers(
        offset_dims=(), collapsed_slice_dims=(0,), start_index_map=(0,))
    return lax.gather(vec, idx, dnums, (1,),
                      mode=lax.GatherScatterMode.PROMISE_IN_BOUNDS)


def _sc_body(table_hbm, idx_hbm, feat_out, cent_out,
             idx_blk, rows0, rows1, self0, self1, feat0, feat1, att_v, att_w,
             pay0, pay1, cent_sh,
             sg0, sg1, ss0, ss1, sf0, sf1, sp0, sp1):
    c_id = lax.axis_index("c")
    s_id = lax.axis_index("s")
    wid = c_id * _NS + s_id
    base_pt = wid * _PPW
    iota16 = lax.iota(jnp.int32, _L)
    zeros16 = jnp.zeros((_L,), jnp.float32)

    for g in range(_G, _L):
        att_v[g, pl.ds(0, _L)] = zeros16
        att_w[g, pl.ds(0, _L)] = zeros16

    def _zb(i, _):
        pay0[i, :] = zeros16
        return 0
    lax.fori_loop(0, _IDXN, _zb, 0)

    rows_per_sub = _NPTS // _NS

    def _zc(i, _):
        pltpu.sync_copy(pay0, cent_sh.at[pl.ds(s_id * rows_per_sub + i * _IDXN, _IDXN)])
        return 0
    lax.fori_loop(0, rows_per_sub // _IDXN, _zc, 0)
    plsc.subcore_barrier()

    def _issue(bi, cl, rows, sg, selfb, ss):
        ci = bi * _BCH + cl
        pltpu.async_copy(table_hbm.at[idx_blk.at[cl]], rows, sg)
        pltpu.async_copy(table_hbm.at[pl.ds(base_pt + ci * _CHUNK, _CHUNK)], selfb, ss)

    def _wait_in(rows, sg, selfb, ss):
        pltpu.make_async_copy(table_hbm.at[pl.ds(0, _IDXN)], rows, sg).wait()
        pltpu.make_async_copy(table_hbm.at[pl.ds(0, _CHUNK)], selfb, ss).wait()

    def _compute(bi, cl, rows, selfb, featb, payb, sf, sp):
        ci = bi * _BCH + cl
        base = base_pt + ci * _CHUNK

        @pl.when(ci >= 2)
        def _():
            pltpu.make_async_copy(featb, feat_out.at[pl.ds(base_pt, _CHUNK)], sf).wait()
            pltpu.make_async_copy(payb, cent_sh.at[idx_blk.at[cl]], sp).wait()

        perms = [(iota16 + sh) & (_L - 1) for sh in (8, 4, 2, 1)]

        def _pt(p, _):
            row0 = p * _K
            accs = []
            for g in range(_G):
                goff = g * _D
                qv = selfb[p, pl.ds(goff, _D)]
                acc = zeros16
                for kk in range(_K):
                    kv = rows[row0 + kk, pl.ds(goff, _D)]
                    s = _tree_sum(qv * kv, perms)
                    acc = jnp.where(iota16 == kk, s, acc)
                accs.append(acc)
            atts = []
            for g in range(_G):
                acc = accs[g]
                m = _tree_max(acc, perms)
                e = jnp.exp(acc - m)
                a = e / _tree_sum(e, perms)
                att_v[g, pl.ds(0, _L)] = a
                atts.append(a)
            for g in range(_G):
                goff = g * _D
                a = atts[g]
                fas = [zeros16, zeros16]
                for kk in range(_K):
                    vv = rows[row0 + kk, pl.ds(_C + goff, _D)]
                    fas[kk % 2] = fas[kk % 2] + _bcast_lane(a, kk) * vv
                featb[p, pl.ds(goff, _D)] = fas[0] + fas[1]
            for kk in range(_K):
                payb[row0 + kk, :] = plsc.load_gather(
                    att_v, [iota16, jnp.full((_L,), kk, jnp.int32)])
            return 0

        lax.fori_loop(0, _CHUNK, _pt, 0)

        pltpu.async_copy(featb, feat_out.at[pl.ds(base, _CHUNK)], sf)
        pltpu.async_copy(payb, cent_sh.at[idx_blk.at[cl]], sp, add=True)

    def _block(bi, _):
        pltpu.sync_copy(idx_hbm.at[wid, bi], idx_blk)
        _issue(bi, 0, rows0, sg0, self0, ss0)

        def _pair(j, _):
            cl0 = 2 * j
            _issue(bi, cl0 + 1, rows1, sg1, self1, ss1)
            _wait_in(rows0, sg0, self0, ss0)
            _compute(bi, cl0, rows0, self0, feat0, pay0, sf0, sp0)

            @pl.when(j + 1 < _BCH // 2)
            def _():
                _issue(bi, cl0 + 2, rows0, sg0, self0, ss0)

            _wait_in(rows1, sg1, self1, ss1)
            _compute(bi, cl0 + 1, rows1, self1, feat1, pay1, sf1, sp1)
            return 0

        lax.fori_loop(0, _BCH // 2, _pair, 0)
        return 0

    lax.fori_loop(0, _NBLK, _block, 0)

    pltpu.make_async_copy(feat0, feat_out.at[pl.ds(base_pt, _CHUNK)], sf0).wait()
    pltpu.make_async_copy(pay0, cent_sh.at[idx_blk.at[0]], sp0).wait()
    pltpu.make_async_copy(feat1, feat_out.at[pl.ds(base_pt, _CHUNK)], sf1).wait()
    pltpu.make_async_copy(pay1, cent_sh.at[idx_blk.at[0]], sp1).wait()

    plsc.subcore_barrier()
    pltpu.sync_copy(cent_sh.at[pl.ds(s_id * rows_per_sub, rows_per_sub)],
                    cent_out.at[c_id, pl.ds(s_id * rows_per_sub, rows_per_sub)])


@jax.jit
def _sc_attention(table, idx_grp):
    mesh = plsc.VectorSubcoreMesh(core_axis_name="c", subcore_axis_name="s")
    call = pl.kernel(
        _sc_body,
        out_type=(
            jax.ShapeDtypeStruct((_NPTS, _C), jnp.float32),
            jax.ShapeDtypeStruct((_NC, _NPTS, _L), jnp.float32),
        ),
        mesh=mesh,
        compiler_params=pltpu.CompilerParams(
            use_tc_tiling_on_sc=False, needs_layout_passes=False),
        scratch_types=[
            pltpu.VMEM((_BCH, _IDXN), jnp.int32),
            pltpu.VMEM((_IDXN, _ROWW), jnp.float32),
            pltpu.VMEM((_IDXN, _ROWW), jnp.float32),
            pltpu.VMEM((_CHUNK, _ROWW), jnp.float32),
            pltpu.VMEM((_CHUNK, _ROWW), jnp.float32),
            pltpu.VMEM((_CHUNK, _C), jnp.float32),
            pltpu.VMEM((_CHUNK, _C), jnp.float32),
            pltpu.VMEM((_L, _L + 1), jnp.float32),
            pltpu.VMEM((_L, _L + 1), jnp.float32),
            pltpu.VMEM((_IDXN, _L), jnp.float32),
            pltpu.VMEM((_IDXN, _L), jnp.float32),
            pltpu.VMEM_SHARED((_NPTS, _L), jnp.float32),
            pltpu.SemaphoreType.DMA,
            pltpu.SemaphoreType.DMA,
            pltpu.SemaphoreType.DMA,
            pltpu.SemaphoreType.DMA,
            pltpu.SemaphoreType.DMA,
            pltpu.SemaphoreType.DMA,
            pltpu.SemaphoreType.DMA,
            pltpu.SemaphoreType.DMA,
        ],
    )
    return call(table, idx_grp)


def kernel(query_xyz, support_xyz, query_mask, support_mask, queryandkey, value, attention_centrality, idx_knn):
    table = (jnp.concatenate([queryandkey, value], axis=1)
             .transpose(0, 2, 1).reshape(_NPTS, _ROWW))
    offs = (jnp.arange(_B, dtype=jnp.int32) * _N)[:, None, None]
    idx_grp = (idx_knn.astype(jnp.int32) + offs).reshape(_NW, _NBLK, _BCH, _IDXN)

    feat_rows, cent_parts = _sc_attention(table, idx_grp)

    feat = feat_rows.reshape(_B, _N, _C).transpose(0, 2, 1)
    cent = cent_parts.sum(0).reshape(_B, _N, _L)[:, :, :_G].transpose(0, 2, 1)
    return feat, cent


# speedup vs baseline: 1.2488x; 1.2488x over previous
import functools

import jax
import jax.numpy as jnp
from jax import lax
from jax.experimental import pallas as pl
from jax.experimental.pallas import tpu as pltpu
from jax.experimental.pallas import tpu_sc as plsc

_B, _C, _N, _K, _G = 2, 144, 16384, 16, 9
_D = _C // _G
_ROWW = 2 * _C
_NPTS = _B * _N
_NC, _NS, _L = 2, 16, 16
_NW = _NC * _NS
_PPW = _NPTS // _NW
_CHUNK = 8
_NCH = _PPW // _CHUNK
_IDXN = _CHUNK * _K
_BCH = 16
_NBLK = _NCH // _BCH


def _perm(vec, idx_const):
    dnums = lax.GatherDimensionNumbers(
        offset_dims=(), collapsed_slice_dims=(0,), start_index_map=(0,))
    return lax.gather(vec, idx_const[:, None], dnums, (1,),
                      mode=lax.GatherScatterMode.PROMISE_IN_BOUNDS)


def _tree_max(v, perms):
    for idxc in perms:
        v = jnp.maximum(v, _perm(v, idxc))
    return v


def _tree_sum(v, perms):
    for idxc in perms:
        v = v + _perm(v, idxc)
    return v


def _bcast_lane(vec, lane):
    idx = jnp.full((_L, 1), lane, jnp.int32)
    dnums = lax.GatherDimensionNumbers(
        offset_dims=(), collapsed_slice_dims=(0,), start_index_map=(0,))
    return lax.gather(vec, idx, dnums, (1,),
                      mode=lax.GatherScatterMode.PROMISE_IN_BOUNDS)


def _sc_body(table_hbm, idx_hbm, feat_out, cent_out,
             idx_blk, rows0, rows1, self0, self1, feat0, feat1, att_v, att_w,
             pay0, pay1, cent_sh,
             sg0, sg1, ss0, ss1, sf0, sf1, sp0, sp1):
    c_id = lax.axis_index("c")
    s_id = lax.axis_index("s")
    wid = c_id * _NS + s_id
    base_pt = wid * _PPW
    iota16 = lax.iota(jnp.int32, _L)
    zeros16 = jnp.zeros((_L,), jnp.float32)

    for g in range(_G, _L):
        att_v[g, pl.ds(0, _L)] = zeros16
        att_w[g, pl.ds(0, _L)] = zeros16

    def _zb(i, _):
        pay0[i, :] = zeros16
        return 0
    lax.fori_loop(0, _IDXN, _zb, 0)

    rows_per_sub = _NPTS // _NS

    def _zc(i, _):
        pltpu.sync_copy(pay0, cent_sh.at[pl.ds(s_id * rows_per_sub + i * _IDXN, _IDXN)])
        return 0
    lax.fori_loop(0, rows_per_sub // _IDXN, _zc, 0)
    plsc.subcore_barrier()

    def _issue(bi, cl, rows, sg, selfb, ss):
        ci = bi * _BCH + cl
        pltpu.async_copy(table_hbm.at[idx_blk.at[cl]], rows, sg)
        pltpu.async_copy(table_hbm.at[pl.ds(base_pt + ci * _CHUNK, _CHUNK)], selfb, ss)

    def _wait_in(rows, sg, selfb, ss):
        pltpu.make_async_copy(table_hbm.at[pl.ds(0, _IDXN)], rows, sg).wait()
        pltpu.make_async_copy(table_hbm.at[pl.ds(0, _CHUNK)], selfb, ss).wait()

    def _compute(bi, cl, rows, selfb, featb, payb, sf, sp):
        ci = bi * _BCH + cl
        base = base_pt + ci * _CHUNK

        @pl.when(ci >= 2)
        def _():
            pltpu.make_async_copy(featb, feat_out.at[pl.ds(base_pt, _CHUNK)], sf).wait()
            pltpu.make_async_copy(payb, cent_sh.at[idx_blk.at[cl]], sp).wait()

        perms = [(iota16 + sh) & (_L - 1) for sh in (8, 4, 2, 1)]

        def _pt(p, _):
            row0 = p * _K
            accs = []
            for g in range(_G):
                goff = g * _D
                qv = selfb[p, pl.ds(goff, _D)]
                acc = zeros16
                for kk in range(_K):
                    kv = rows[row0 + kk, pl.ds(goff, _D)]
                    s = jnp.sum(qv * kv)
                    acc = jnp.where(iota16 == kk, s, acc)
                accs.append(acc)
            atts = []
            for g in range(_G):
                acc = accs[g]
                m = _tree_max(acc, perms)
                e = jnp.exp(acc - m)
                a = e / _tree_sum(e, perms)
                att_v[g, pl.ds(0, _L)] = a
                atts.append(a)
            for g in range(_G):
                goff = g * _D
                a = atts[g]
                fas = [zeros16, zeros16]
                for kk in range(_K):
                    vv = rows[row0 + kk, pl.ds(_C + goff, _D)]
                    fas[kk % 2] = fas[kk % 2] + _bcast_lane(a, kk) * vv
                featb[p, pl.ds(goff, _D)] = fas[0] + fas[1]
            for kk in range(_K):
                payb[row0 + kk, :] = plsc.load_gather(
                    att_v, [iota16, jnp.full((_L,), kk, jnp.int32)])
            return 0

        lax.fori_loop(0, _CHUNK, _pt, 0)

        pltpu.async_copy(featb, feat_out.at[pl.ds(base, _CHUNK)], sf)
        pltpu.async_copy(payb, cent_sh.at[idx_blk.at[cl]], sp, add=True)

    def _block(bi, _):
        pltpu.sync_copy(idx_hbm.at[wid, bi], idx_blk)
        _issue(bi, 0, rows0, sg0, self0, ss0)

        def _pair(j, _):
            cl0 = 2 * j
            _issue(bi, cl0 + 1, rows1, sg1, self1, ss1)
            _wait_in(rows0, sg0, self0, ss0)
            _compute(bi, cl0, rows0, self0, feat0, pay0, sf0, sp0)

            @pl.when(j + 1 < _BCH // 2)
            def _():
                _issue(bi, cl0 + 2, rows0, sg0, self0, ss0)

            _wait_in(rows1, sg1, self1, ss1)
            _compute(bi, cl0 + 1, rows1, self1, feat1, pay1, sf1, sp1)
            return 0

        lax.fori_loop(0, _BCH // 2, _pair, 0)
        return 0

    lax.fori_loop(0, _NBLK, _block, 0)

    pltpu.make_async_copy(feat0, feat_out.at[pl.ds(base_pt, _CHUNK)], sf0).wait()
    pltpu.make_async_copy(pay0, cent_sh.at[idx_blk.at[0]], sp0).wait()
    pltpu.make_async_copy(feat1, feat_out.at[pl.ds(base_pt, _CHUNK)], sf1).wait()
    pltpu.make_async_copy(pay1, cent_sh.at[idx_blk.at[0]], sp1).wait()

    plsc.subcore_barrier()
    pltpu.sync_copy(cent_sh.at[pl.ds(s_id * rows_per_sub, rows_per_sub)],
                    cent_out.at[c_id, pl.ds(s_id * rows_per_sub, rows_per_sub)])


@jax.jit
def _sc_attention(table, idx_grp):
    mesh = plsc.VectorSubcoreMesh(core_axis_name="c", subcore_axis_name="s")
    call = pl.kernel(
        _sc_body,
        out_type=(
            jax.ShapeDtypeStruct((_NPTS, _C), jnp.float32),
            jax.ShapeDtypeStruct((_NC, _NPTS, _L), jnp.float32),
        ),
        mesh=mesh,
        compiler_params=pltpu.CompilerParams(
            use_tc_tiling_on_sc=False, needs_layout_passes=False),
        scratch_types=[
            pltpu.VMEM((_BCH, _IDXN), jnp.int32),
            pltpu.VMEM((_IDXN, _ROWW), jnp.float32),
            pltpu.VMEM((_IDXN, _ROWW), jnp.float32),
            pltpu.VMEM((_CHUNK, _ROWW), jnp.float32),
            pltpu.VMEM((_CHUNK, _ROWW), jnp.float32),
            pltpu.VMEM((_CHUNK, _C), jnp.float32),
            pltpu.VMEM((_CHUNK, _C), jnp.float32),
            pltpu.VMEM((_L, _L + 1), jnp.float32),
            pltpu.VMEM((_L, _L + 1), jnp.float32),
            pltpu.VMEM((_IDXN, _L), jnp.float32),
            pltpu.VMEM((_IDXN, _L), jnp.float32),
            pltpu.VMEM_SHARED((_NPTS, _L), jnp.float32),
            pltpu.SemaphoreType.DMA,
            pltpu.SemaphoreType.DMA,
            pltpu.SemaphoreType.DMA,
            pltpu.SemaphoreType.DMA,
            pltpu.SemaphoreType.DMA,
            pltpu.SemaphoreType.DMA,
            pltpu.SemaphoreType.DMA,
            pltpu.SemaphoreType.DMA,
        ],
    )
    return call(table, idx_grp)


def kernel(query_xyz, support_xyz, query_mask, support_mask, queryandkey, value, attention_centrality, idx_knn):
    table = (jnp.concatenate([queryandkey, value], axis=1)
             .transpose(0, 2, 1).reshape(_NPTS, _ROWW))
    offs = (jnp.arange(_B, dtype=jnp.int32) * _N)[:, None, None]
    idx_grp = (idx_knn.astype(jnp.int32) + offs).reshape(_NW, _NBLK, _BCH, _IDXN)

    feat_rows, cent_parts = _sc_attention(table, idx_grp)

    feat = feat_rows.reshape(_B, _N, _C).transpose(0, 2, 1)
    cent = cent_parts.sum(0).reshape(_B, _N, _L)[:, :, :_G].transpose(0, 2, 1)
    return feat, cent
